# baseline (speedup 1.0000x reference)
import functools

import jax
import jax.numpy as jnp
from jax.experimental import pallas as pl


def _encoder_kernel(feats_ref, s_ref, cst_ref, t_ref, wb1t_ref, bb1_ref,
                    wbeh_ref, bz_ref, wf2t_ref, bf2_ref, out_ref):
    f = feats_ref[...]
    R = f.shape[0]

    behavior = f[:, 7:15]
    h = jnp.maximum(
        jnp.dot(behavior, wb1t_ref[...],
                preferred_element_type=jnp.float32) + bb1_ref[...], 0.0)

    bc = jnp.dot(f[:, 0:8], s_ref[...], preferred_element_type=jnp.float32)
    mh = (bc == cst_ref[...]).astype(jnp.float32)

    z = (jnp.dot(mh, t_ref[...], preferred_element_type=jnp.float32)
         + jnp.dot(h, wbeh_ref[...], preferred_element_type=jnp.float32)
         + bz_ref[...])
    g = jnp.maximum(z, 0.0)
    out = jnp.dot(g, wf2t_ref[...], preferred_element_type=jnp.float32)
    out_ref[...] = out + bf2_ref[...]


@jax.jit
def kernel(player_features, hand_tab, suit_tab, bid_tab, role_tab,
           Wb1, bb1, Wb2, bb2, Wf1, bf1, Wf2, bf2):
    B, P, D = player_features.shape[0], player_features.shape[1], Wf1.shape[0]
    N = B * P
    feats = player_features.reshape(N, 15)

    Wf1T = Wf1.T
    t_hand = hand_tab @ Wf1T[0:16]
    t_suit = (0.25 * suit_tab) @ Wf1T[16:32]
    t_bid = bid_tab @ Wf1T[32:48]
    t_role = role_tab @ Wf1T[48:64]
    def pad_rows(t, n):
        return jnp.concatenate(
            [t, jnp.zeros((n - t.shape[0], D), jnp.float32)], axis=0)
    t128 = jnp.concatenate(
        [pad_rows(t_hand, 16), pad_rows(t_suit, 16), pad_rows(t_suit, 16),
         pad_rows(t_suit, 16), pad_rows(t_suit, 16), pad_rows(t_bid, 32),
         pad_rows(t_role, 16)], axis=0)

    seg_starts = jnp.array([0, 16, 32, 48, 64, 80, 112], jnp.int32)
    seg_widths = jnp.array([16, 16, 16, 16, 16, 32, 16], jnp.int32)
    lane = jnp.arange(128, dtype=jnp.int32)
    in_seg = ((lane[None, :] >= seg_starts[:, None])
              & (lane[None, :] < (seg_starts + seg_widths)[:, None]))
    s_mat = jnp.concatenate(
        [in_seg.astype(jnp.float32),
         jnp.zeros((1, 128), jnp.float32)], axis=0)
    seg_of_lane = (lane[None, :] >= seg_starts[:, None]).astype(
        jnp.int32).sum(axis=0) - 1
    cst = (lane - seg_starts[seg_of_lane]).astype(
        jnp.float32).reshape(1, 128)

    w_beh = Wb2.T @ Wf1T[64:128]
    bz = (bf1 + bb2 @ Wf1T[64:128]).reshape(1, D)
    wb1t = Wb1.T
    wf2t = Wf2.T

    R = 16384
    grid = (N // R,)
    out = pl.pallas_call(
        _encoder_kernel,
        grid=grid,
        in_specs=[
            pl.BlockSpec((R, 15), lambda i: (i, 0)),
            pl.BlockSpec((8, 128), lambda i: (0, 0)),
            pl.BlockSpec((1, 128), lambda i: (0, 0)),
            pl.BlockSpec((128, 128), lambda i: (0, 0)),
            pl.BlockSpec((8, 64), lambda i: (0, 0)),
            pl.BlockSpec((1, 64), lambda i: (0, 0)),
            pl.BlockSpec((64, 128), lambda i: (0, 0)),
            pl.BlockSpec((1, 128), lambda i: (0, 0)),
            pl.BlockSpec((128, 128), lambda i: (0, 0)),
            pl.BlockSpec((1, 128), lambda i: (0, 0)),
        ],
        out_specs=pl.BlockSpec((R, 128), lambda i: (i, 0)),
        out_shape=jax.ShapeDtypeStruct((N, D), jnp.float32),
    )(feats, s_mat, cst, t128, wb1t, bb1.reshape(1, 64), w_beh, bz, wf2t,
      bf2.reshape(1, D))
    return out.reshape(B, P, D)


# speedup vs baseline: 1.4252x; 1.4252x over previous
import jax
import jax.numpy as jnp
from jax import lax
from jax.experimental import pallas as pl
from jax.experimental.pallas import tpu as pltpu

_SEG_STARTS = (0, 16, 32, 48, 64, 80, 112)
_SEG_ENDS = (16, 32, 48, 64, 80, 112, 128)


def _dotT(a, b):
    return lax.dot_general(a, b, (((1,), (1,)), ((), ())),
                           preferred_element_type=jnp.float32)


def _encoder_kernel(feats_ref, hand_ref, suit_ref, bid_ref, role_ref,
                    wb1_ref, bb1_ref, wb2_ref, bb2_ref, wf1_ref, bf1_ref,
                    wf2_ref, bf2_ref, out_ref, t128_s, wbeh_s, bz_s):
    @pl.when(pl.program_id(0) == 0)
    def _fold():
        wf1 = wf1_ref[...]
        t_hand = _dotT(hand_ref[...], wf1[:, 0:16])
        t_suit = _dotT(0.25 * suit_ref[...], wf1[:, 16:32])
        t_bid = _dotT(bid_ref[...], wf1[:, 32:48])
        t_role = _dotT(role_ref[...], wf1[:, 48:64])
        t128_s[...] = jnp.zeros_like(t128_s)
        t128_s[0:9, :] = t_hand
        t128_s[16:25, :] = t_suit
        t128_s[32:41, :] = t_suit
        t128_s[48:57, :] = t_suit
        t128_s[64:73, :] = t_suit
        t128_s[80:109, :] = t_bid
        t128_s[112:116, :] = t_role
        wf1_beh = wf1[:, 64:128]
        wbeh_s[...] = lax.dot_general(
            wb2_ref[...], wf1_beh, (((0,), (1,)), ((), ())),
            preferred_element_type=jnp.float32)
        bz_s[...] = bf1_ref[...] + _dotT(bb2_ref[...], wf1_beh)

    f = feats_ref[...]

    h = jnp.maximum(_dotT(f[:, 7:15], wb1_ref[...]) + bb1_ref[...], 0.0)

    row8 = lax.broadcasted_iota(jnp.int32, (8, 128), 0)
    lane8 = lax.broadcasted_iota(jnp.int32, (8, 128), 1)
    smat = jnp.zeros((8, 128), jnp.float32)
    for c, (s, e) in enumerate(zip(_SEG_STARTS, _SEG_ENDS)):
        smat = smat + jnp.where((row8 == c) & (lane8 >= s) & (lane8 < e),
                                1.0, 0.0)
    lane1 = lax.broadcasted_iota(jnp.int32, (1, 128), 1)
    seg_start = jnp.zeros((1, 128), jnp.int32)
    for s, e in zip(_SEG_STARTS, _SEG_ENDS):
        seg_start = jnp.where((lane1 >= s) & (lane1 < e), s, seg_start)
    cst = (lane1 - seg_start).astype(jnp.float32)

    bc = jnp.dot(f[:, 0:8], smat, preferred_element_type=jnp.float32)
    mh = (bc == cst).astype(jnp.float32)

    z = (jnp.dot(mh, t128_s[...], preferred_element_type=jnp.float32)
         + jnp.dot(h, wbeh_s[...], preferred_element_type=jnp.float32)
         + bz_s[...])
    g = jnp.maximum(z, 0.0)
    out_ref[...] = _dotT(g, wf2_ref[...]) + bf2_ref[...]


@jax.jit
def kernel(player_features, hand_tab, suit_tab, bid_tab, role_tab,
           Wb1, bb1, Wb2, bb2, Wf1, bf1, Wf2, bf2):
    B, P, D = player_features.shape[0], player_features.shape[1], Wf1.shape[0]
    N = B * P
    feats = player_features.reshape(N, 15)

    R = 8192
    grid = (N // R,)

    def full(shape):
        return pl.BlockSpec(shape, lambda i: (0,) * len(shape))

    out = pl.pallas_call(
        _encoder_kernel,
        grid=grid,
        in_specs=[
            pl.BlockSpec((R, 15), lambda i: (i, 0)),
            full((9, 16)), full((9, 16)), full((29, 16)), full((4, 16)),
            full((64, 8)), full((1, 64)), full((64, 64)), full((1, 64)),
            full((128, 128)), full((1, 128)), full((128, 128)),
            full((1, 128)),
        ],
        out_specs=pl.BlockSpec((R, 128), lambda i: (i, 0)),
        out_shape=jax.ShapeDtypeStruct((N, D), jnp.float32),
        scratch_shapes=[
            pltpu.VMEM((128, 128), jnp.float32),
            pltpu.VMEM((64, 128), jnp.float32),
            pltpu.VMEM((1, 128), jnp.float32),
        ],
    )(feats, hand_tab, suit_tab, bid_tab, role_tab, Wb1,
      bb1.reshape(1, 64), Wb2, bb2.reshape(1, 64), Wf1, bf1.reshape(1, 128),
      Wf2, bf2.reshape(1, 128))
    return out.reshape(B, P, D)
